# baseline (speedup 1.0000x reference)
import jax
import jax.numpy as jnp
from jax.experimental import pallas as pl
from jax.experimental.pallas import tpu as pltpu

_TILE = 4096
_UNROLL = 64


def _make_slab_kernel(v_rows, d_pad):
    s = d_pad // 128
    q = 8 // s

    def _kernel(idx_ref, w_ref, o_ref, tslab):
        iota = jax.lax.broadcasted_iota(jnp.int32, (8, 128), 0)

        @pl.when(pl.program_id(0) == 0)
        def _build_slab_table():
            def build(b, carry):
                for j in range(4):
                    r8 = pl.multiple_of(b * 32 + j * 8, 8)
                    src = w_ref[pl.ds(r8, 8), :]
                    d0 = pl.multiple_of((b * 32 + j * 8) * s, 8)
                    for k in range(s):
                        tslab[pl.Slice(d0 + k, 8, s), :] = (
                            src[:, k * 128:(k + 1) * 128])
                return carry

            jax.lax.fori_loop(0, v_rows // 32, build, 0)

        base = pl.program_id(0) * _TILE

        def body(c, carry):
            off = c * _UNROLL
            for g in range(_UNROLL // 8):
                goff = off + g * 8
                slabs = []
                for t in range(8):
                    i4 = pl.multiple_of(idx_ref[base + goff + t], s)
                    slabs.append(tslab[pl.ds(i4, s), :])
                packs = [jnp.concatenate(slabs[q * m:q * m + q], axis=0)
                         for m in range(s)]
                outs = []
                for k in range(s):
                    acc = None
                    for m in range(s):
                        for a in range(q):
                            t_ = q * m + a
                            src_sl = a * s + k
                            r = pltpu.roll(packs[m], (t_ - src_sl) % 8,
                                           axis=0)
                            acc = r if acc is None else jnp.where(
                                iota == t_, r, acc)
                    outs.append(acc)
                val = jnp.concatenate(outs, axis=1)
                o_ref[pl.ds(pl.multiple_of(goff, 8), 8), :] = val
            return carry

        jax.lax.fori_loop(0, _TILE // _UNROLL, body, 0)

    return _kernel


def _round_up(n, m):
    return ((n + m - 1) // m) * m


def kernel(x, weight):
    B, S = x.shape
    V, D = weight.shape
    N = B * S

    D_pad = _round_up(D, 128)
    if D_pad != D:
        weight = jnp.pad(weight, ((0, 0), (0, D_pad - D)))
    s = D_pad // 128

    idx = jnp.clip(x.reshape(N).astype(jnp.int32), 0, V - 1)
    N_pad = _round_up(N, _TILE)
    if N_pad != N:
        idx = jnp.pad(idx, (0, N_pad - N))
    idx = idx * s

    out = pl.pallas_call(
        _make_slab_kernel(V, D_pad),
        out_shape=jax.ShapeDtypeStruct((N_pad, D_pad), weight.dtype),
        grid_spec=pltpu.PrefetchScalarGridSpec(
            num_scalar_prefetch=1,
            grid=(N_pad // _TILE,),
            in_specs=[
                pl.BlockSpec((V, D_pad), lambda i, ids: (0, 0)),
            ],
            out_specs=pl.BlockSpec((_TILE, D_pad), lambda i, ids: (i, 0)),
            scratch_shapes=[pltpu.VMEM((V * s, 128), weight.dtype)],
        ),
        compiler_params=pltpu.CompilerParams(
            dimension_semantics=("arbitrary",),
            vmem_limit_bytes=56 * 1024 * 1024,
        ),
    )(idx, weight)

    return out[:N, :D].reshape(B, S, D)


# speedup vs baseline: 1.0376x; 1.0376x over previous
import jax
import jax.numpy as jnp
from jax.experimental import pallas as pl
from jax.experimental.pallas import tpu as pltpu

_TILE = 2048
_UNROLL = 64


def _make_slab_kernel(v_rows, d_pad):
    s = d_pad // 128
    q = 8 // s

    def _kernel(idx_ref, w_ref, o_ref, tslab):
        iota = jax.lax.broadcasted_iota(jnp.int32, (8, 128), 0)

        @pl.when(pl.program_id(0) == 0)
        def _build_slab_table():
            def build(b, carry):
                for j in range(4):
                    r8 = pl.multiple_of(b * 32 + j * 8, 8)
                    src = w_ref[pl.ds(r8, 8), :]
                    d0 = pl.multiple_of((b * 32 + j * 8) * s, 8)
                    for k in range(s):
                        tslab[pl.Slice(d0 + k, 8, s), :] = (
                            src[:, k * 128:(k + 1) * 128])
                return carry

            jax.lax.fori_loop(0, v_rows // 32, build, 0)

        base = pl.program_id(0) * _TILE

        def body(c, carry):
            off = c * _UNROLL
            for g in range(_UNROLL // 8):
                goff = off + g * 8
                slabs = []
                for t in range(8):
                    i4 = pl.multiple_of(idx_ref[base + goff + t], s)
                    slabs.append(tslab[pl.ds(i4, s), :])
                packs = [jnp.concatenate(slabs[q * m:q * m + q], axis=0)
                         for m in range(s)]
                outs = []
                for k in range(s):
                    acc = None
                    for m in range(s):
                        for a in range(q):
                            t_ = q * m + a
                            src_sl = a * s + k
                            r = pltpu.roll(packs[m], (t_ - src_sl) % 8,
                                           axis=0)
                            acc = r if acc is None else jnp.where(
                                iota == t_, r, acc)
                    outs.append(acc)
                val = jnp.concatenate(outs, axis=1)
                o_ref[pl.ds(pl.multiple_of(goff, 8), 8), :] = val
            return carry

        jax.lax.fori_loop(0, _TILE // _UNROLL, body, 0)

    return _kernel


def _round_up(n, m):
    return ((n + m - 1) // m) * m


def kernel(x, weight):
    B, S = x.shape
    V, D = weight.shape
    N = B * S

    D_pad = _round_up(D, 128)
    if D_pad != D:
        weight = jnp.pad(weight, ((0, 0), (0, D_pad - D)))
    s = D_pad // 128

    idx = jnp.clip(x.reshape(N).astype(jnp.int32), 0, V - 1)
    N_pad = _round_up(N, _TILE)
    if N_pad != N:
        idx = jnp.pad(idx, (0, N_pad - N))
    idx = idx * s

    out = pl.pallas_call(
        _make_slab_kernel(V, D_pad),
        out_shape=jax.ShapeDtypeStruct((N_pad, D_pad), weight.dtype),
        grid_spec=pltpu.PrefetchScalarGridSpec(
            num_scalar_prefetch=1,
            grid=(N_pad // _TILE,),
            in_specs=[
                pl.BlockSpec((V, D_pad), lambda i, ids: (0, 0)),
            ],
            out_specs=pl.BlockSpec((_TILE, D_pad), lambda i, ids: (i, 0)),
            scratch_shapes=[pltpu.VMEM((V * s, 128), weight.dtype)],
        ),
        compiler_params=pltpu.CompilerParams(
            dimension_semantics=("arbitrary",),
            vmem_limit_bytes=48 * 1024 * 1024,
        ),
    )(idx, weight)

    return out[:N, :D].reshape(B, S, D)


# speedup vs baseline: 1.0861x; 1.0467x over previous
import jax
import jax.numpy as jnp
from jax.experimental import pallas as pl
from jax.experimental.pallas import tpu as pltpu

_TILE = 2048
_UNROLL = 64


def _make_slab_kernel(v_rows, d_pad):
    s = d_pad // 128
    q = 8 // s

    def _kernel(idx_ref, w_hbm, o_ref, tslab, wtmp, sem):
        iota = jax.lax.broadcasted_iota(jnp.int32, (8, 128), 0)

        @pl.when(pl.program_id(0) == 0)
        def _build_slab_table():
            n_chunks = 8
            rows = v_rows // n_chunks
            copies = [
                pltpu.make_async_copy(
                    w_hbm.at[pl.ds(c * rows, rows), :],
                    wtmp.at[pl.ds(c * rows, rows), :],
                    sem.at[c],
                )
                for c in range(n_chunks)
            ]
            for cp in copies:
                cp.start()
            for c in range(n_chunks):
                copies[c].wait()

                def build(b, carry):
                    for j in range(4):
                        r8 = pl.multiple_of(c * rows + b * 32 + j * 8, 8)
                        srcv = wtmp[pl.ds(r8, 8), :]
                        d0 = pl.multiple_of((c * rows + b * 32 + j * 8) * s, 8)
                        for k in range(s):
                            tslab[pl.Slice(d0 + k, 8, s), :] = (
                                srcv[:, k * 128:(k + 1) * 128])
                    return carry

                jax.lax.fori_loop(0, rows // 32, build, 0)

        base = pl.program_id(0) * _TILE

        def body(c, carry):
            off = c * _UNROLL
            for g in range(_UNROLL // 8):
                goff = off + g * 8
                slabs = []
                for t in range(8):
                    i4 = pl.multiple_of(idx_ref[base + goff + t], s)
                    slabs.append(tslab[pl.ds(i4, s), :])
                packs = [jnp.concatenate(slabs[q * m:q * m + q], axis=0)
                         for m in range(s)]
                outs = []
                for k in range(s):
                    acc = None
                    for m in range(s):
                        for a in range(q):
                            t_ = q * m + a
                            src_sl = a * s + k
                            r = pltpu.roll(packs[m], (t_ - src_sl) % 8,
                                           axis=0)
                            acc = r if acc is None else jnp.where(
                                iota == t_, r, acc)
                    outs.append(acc)
                val = jnp.concatenate(outs, axis=1)
                o_ref[pl.ds(pl.multiple_of(goff, 8), 8), :] = val
            return carry

        jax.lax.fori_loop(0, _TILE // _UNROLL, body, 0)

    return _kernel


def _round_up(n, m):
    return ((n + m - 1) // m) * m


def kernel(x, weight):
    B, S = x.shape
    V, D = weight.shape
    N = B * S

    D_pad = _round_up(D, 128)
    if D_pad != D:
        weight = jnp.pad(weight, ((0, 0), (0, D_pad - D)))
    s = D_pad // 128

    idx = jnp.clip(x.reshape(N).astype(jnp.int32), 0, V - 1)
    N_pad = _round_up(N, _TILE)
    if N_pad != N:
        idx = jnp.pad(idx, (0, N_pad - N))
    idx = idx * s

    out = pl.pallas_call(
        _make_slab_kernel(V, D_pad),
        out_shape=jax.ShapeDtypeStruct((N_pad, D_pad), weight.dtype),
        grid_spec=pltpu.PrefetchScalarGridSpec(
            num_scalar_prefetch=1,
            grid=(N_pad // _TILE,),
            in_specs=[
                pl.BlockSpec(memory_space=pl.ANY),
            ],
            out_specs=pl.BlockSpec((_TILE, D_pad), lambda i, ids: (i, 0)),
            scratch_shapes=[
                pltpu.VMEM((V * s, 128), weight.dtype),
                pltpu.VMEM((V, D_pad), weight.dtype),
                pltpu.SemaphoreType.DMA((8,)),
            ],
        ),
        compiler_params=pltpu.CompilerParams(
            dimension_semantics=("arbitrary",),
            vmem_limit_bytes=48 * 1024 * 1024,
        ),
    )(idx, weight)

    return out[:N, :D].reshape(B, S, D)
